# baseline (speedup 1.0000x reference)
import functools

import jax
import jax.numpy as jnp
from jax import lax
from jax.experimental import pallas as pl
from jax.experimental.pallas import tpu as pltpu
from jax.experimental.pallas import tpu_sc as plsc

VOCAB = 100000
EMBED_DIM = 300
EMBED_PAD = 304
BATCH = 4096
SEQ = 50

N_IDX = BATCH * SEQ
NUM_WORKERS = 32
PER_WORKER = N_IDX // NUM_WORKERS
CHUNK = 128
N_CHUNKS = PER_WORKER // CHUNK

_mesh = plsc.VectorSubcoreMesh(core_axis_name="c", subcore_axis_name="s")


@functools.partial(
    pl.kernel,
    mesh=_mesh,
    out_type=jax.ShapeDtypeStruct((N_IDX, EMBED_PAD), jnp.float32),
    scratch_types=[
        pltpu.VMEM((CHUNK,), jnp.int32),
        pltpu.VMEM((CHUNK, EMBED_PAD), jnp.float32),
        pltpu.SemaphoreType.DMA,
    ],
    compiler_params=pltpu.CompilerParams(use_tc_tiling_on_sc=False),
)
def _gather_sc(idx_hbm, table_hbm, out_hbm, idx_v, rows_v, sem):
    wid = lax.axis_index("s") * 2 + lax.axis_index("c")
    base = wid * PER_WORKER

    def chunk_body(j, carry):
        off = base + j * CHUNK
        pltpu.sync_copy(idx_hbm.at[pl.ds(off, CHUNK)], idx_v)
        pltpu.async_copy(table_hbm.at[idx_v], rows_v, sem).wait()
        pltpu.sync_copy(rows_v, out_hbm.at[pl.ds(off, CHUNK)])
        return carry

    lax.fori_loop(0, N_CHUNKS, chunk_body, 0)


def kernel(batch, table):
    flat = batch.reshape(N_IDX)
    tpad = jnp.pad(table, ((0, 0), (0, EMBED_PAD - EMBED_DIM)))
    out = _gather_sc(flat, tpad)
    return out[:, :EMBED_DIM].reshape(BATCH, SEQ, EMBED_DIM)


# speedup vs baseline: 1.0951x; 1.0951x over previous
import functools

import jax
import jax.numpy as jnp
from jax import lax
from jax.experimental import pallas as pl
from jax.experimental.pallas import tpu as pltpu
from jax.experimental.pallas import tpu_sc as plsc

VOCAB = 100000
EMBED_DIM = 300
EMBED_PAD = 304
BATCH = 4096
SEQ = 50

N_IDX = BATCH * SEQ
NUM_WORKERS = 32
PER_WORKER = N_IDX // NUM_WORKERS
CHUNK = 128
N_CHUNKS = PER_WORKER // CHUNK

_mesh = plsc.VectorSubcoreMesh(core_axis_name="c", subcore_axis_name="s")


@functools.partial(
    pl.kernel,
    mesh=_mesh,
    out_type=jax.ShapeDtypeStruct((N_IDX, EMBED_PAD), jnp.float32),
    scratch_types=[
        pltpu.VMEM((CHUNK,), jnp.int32),
        pltpu.VMEM((CHUNK, EMBED_PAD), jnp.float32),
        pltpu.SemaphoreType.DMA,
    ],
    compiler_params=pltpu.CompilerParams(use_tc_tiling_on_sc=False),
)
def _gather_sc(idx_hbm, table_hbm, out_hbm, idx_v, rows_v, sem):
    wid = lax.axis_index("s") * 2 + lax.axis_index("c")
    base = wid * PER_WORKER

    def chunk_body(j, carry):
        off = base + j * CHUNK
        pltpu.sync_copy(idx_hbm.at[pl.ds(off, CHUNK)], idx_v)
        pltpu.async_copy(table_hbm.at[idx_v], rows_v, sem).wait()
        pltpu.sync_copy(rows_v, out_hbm.at[pl.ds(off, CHUNK)])
        return carry

    lax.fori_loop(0, N_CHUNKS, chunk_body, 0)


_PAD_ROWS = 2000


def _pad_body(t_ref, o_ref):
    o_ref[:, :EMBED_DIM] = t_ref[...]
    o_ref[:, EMBED_DIM:] = jnp.zeros((_PAD_ROWS, EMBED_PAD - EMBED_DIM),
                                     jnp.float32)


_tc_pad = pl.pallas_call(
    _pad_body,
    grid=(VOCAB // _PAD_ROWS,),
    in_specs=[pl.BlockSpec((_PAD_ROWS, EMBED_DIM), lambda i: (i, 0))],
    out_specs=pl.BlockSpec((_PAD_ROWS, EMBED_PAD), lambda i: (i, 0)),
    out_shape=jax.ShapeDtypeStruct((VOCAB, EMBED_PAD), jnp.float32),
)

_SLC_ROWS = 2048


def _slice_body(p_ref, o_ref):
    o_ref[...] = p_ref[:, :EMBED_DIM]


_tc_slice = pl.pallas_call(
    _slice_body,
    grid=(N_IDX // _SLC_ROWS,),
    in_specs=[pl.BlockSpec((_SLC_ROWS, EMBED_PAD), lambda i: (i, 0))],
    out_specs=pl.BlockSpec((_SLC_ROWS, EMBED_DIM), lambda i: (i, 0)),
    out_shape=jax.ShapeDtypeStruct((N_IDX, EMBED_DIM), jnp.float32),
)


def kernel(batch, table):
    flat = batch.reshape(N_IDX)
    tpad = _tc_pad(table)
    outp = _gather_sc(flat, tpad)
    out = _tc_slice(outp)
    return out.reshape(BATCH, SEQ, EMBED_DIM)


# speedup vs baseline: 1.5276x; 1.3950x over previous
import functools

import jax
import jax.numpy as jnp
from jax import lax
from jax.experimental import pallas as pl
from jax.experimental.pallas import tpu as pltpu
from jax.experimental.pallas import tpu_sc as plsc

VOCAB = 100000
EMBED_DIM = 300
EMBED_PAD = 384
BATCH = 4096
SEQ = 50

N_IDX = BATCH * SEQ
NUM_WORKERS = 32
PER_WORKER = N_IDX // NUM_WORKERS
CHUNK = 128
N_CHUNKS = PER_WORKER // CHUNK

_mesh = plsc.VectorSubcoreMesh(core_axis_name="c", subcore_axis_name="s")


@functools.partial(
    pl.kernel,
    mesh=_mesh,
    out_type=jax.ShapeDtypeStruct((N_IDX, EMBED_PAD), jnp.float32),
    scratch_types=[
        pltpu.VMEM((CHUNK,), jnp.int32),
        pltpu.VMEM((CHUNK, EMBED_PAD), jnp.float32),
        pltpu.SemaphoreType.DMA,
    ],
)
def _gather_sc(idx_hbm, table_hbm, out_hbm, idx_v, rows_v, sem):
    wid = lax.axis_index("s") * 2 + lax.axis_index("c")
    base = wid * PER_WORKER

    def chunk_body(j, carry):
        off = base + j * CHUNK
        pltpu.sync_copy(idx_hbm.at[pl.ds(off, CHUNK)], idx_v)
        pltpu.async_copy(table_hbm.at[idx_v], rows_v, sem).wait()
        pltpu.sync_copy(rows_v, out_hbm.at[pl.ds(off, CHUNK)])
        return carry

    lax.fori_loop(0, N_CHUNKS, chunk_body, 0)


_PAD_ROWS = 2000


def _pad_body(t_ref, o_ref):
    o_ref[:, :EMBED_DIM] = t_ref[...]
    o_ref[:, EMBED_DIM:] = jnp.zeros((_PAD_ROWS, EMBED_PAD - EMBED_DIM),
                                     jnp.float32)


_tc_pad = pl.pallas_call(
    _pad_body,
    grid=(VOCAB // _PAD_ROWS,),
    in_specs=[pl.BlockSpec((_PAD_ROWS, EMBED_DIM), lambda i: (i, 0))],
    out_specs=pl.BlockSpec((_PAD_ROWS, EMBED_PAD), lambda i: (i, 0)),
    out_shape=jax.ShapeDtypeStruct((VOCAB, EMBED_PAD), jnp.float32),
)

_SLC_ROWS = 2048


def _slice_body(p_ref, o_ref):
    o_ref[...] = p_ref[:, :EMBED_DIM]


_tc_slice = pl.pallas_call(
    _slice_body,
    grid=(N_IDX // _SLC_ROWS,),
    in_specs=[pl.BlockSpec((_SLC_ROWS, EMBED_PAD), lambda i: (i, 0))],
    out_specs=pl.BlockSpec((_SLC_ROWS, EMBED_DIM), lambda i: (i, 0)),
    out_shape=jax.ShapeDtypeStruct((N_IDX, EMBED_DIM), jnp.float32),
)


def kernel(batch, table):
    flat = batch.reshape(N_IDX)
    tpad = _tc_pad(table)
    outp = _gather_sc(flat, tpad)
    out = _tc_slice(outp)
    return out.reshape(BATCH, SEQ, EMBED_DIM)
